# Initial kernel scaffold; baseline (speedup 1.0000x reference)
import jax
import jax.numpy as jnp
from jax.experimental import pallas as pl


def kernel(structure_ids, dep_levels, ast_depth_ids, sibling_index_ids, node_type_ids, emb_weight, up_proj_weight, component_scales):
    raise NotImplementedError("write your pallas kernel here")


# speedup vs baseline: 3.2606x; 3.2606x over previous
import functools

import jax
import jax.numpy as jnp
from jax import lax
from jax.experimental import pallas as pl
from jax.experimental.pallas import tpu as pltpu
from jax.experimental.pallas import tpu_sc as plsc

_B, _S = 4, 8192
_T = _B * _S
_D = 64
_H = 1024
_VS = (9, 16, 64, 64, 256)
_OFF = (0, 9, 25, 89, 153)
_NCOMP = 5

_NC, _NS = 2, 16
_NW = _NC * _NS
_TW = _T // _NW
_C = 256
_NSUB = _TW // _C
_G = 128
_NG = _NCOMP * _C // _G


def _sc_weighted(ids, emb_weight, scales_b):
    mesh = plsc.VectorSubcoreMesh(core_axis_name="c", subcore_axis_name="s")

    @functools.partial(
        pl.kernel,
        out_type=jax.ShapeDtypeStruct((_T, _D), jnp.float32),
        mesh=mesh,
        scratch_types=[
            pltpu.VMEM((_NCOMP * _C,), jnp.int32),
            pltpu.VMEM((_NCOMP * _C, _D), jnp.float32),
            pltpu.VMEM((_C, _D), jnp.float32),
            pltpu.VMEM((_NCOMP, 16), jnp.float32),
            pltpu.SemaphoreType.DMA,
        ],
        compiler_params=pltpu.CompilerParams(use_tc_tiling_on_sc=False),
    )
    def body(ids_hbm, emb_hbm, scales_hbm, w_hbm, idbuf, rows, wbuf, scv, gsem):
        wid = lax.axis_index("s") * _NC + lax.axis_index("c")
        base = wid * _TW
        pltpu.sync_copy(scales_hbm, scv)
        svec = [scv[c, :] for c in range(_NCOMP)]
        for sub in range(_NSUB):
            tb = base + sub * _C
            for c in range(_NCOMP):
                pltpu.sync_copy(ids_hbm.at[pl.ds(c * _T + tb, _C)],
                                idbuf.at[pl.ds(c * _C, _C)])
            for c in range(_NCOMP):
                lo = c * _C

                def tbody(j, _, lo=lo, c=c):
                    o = lo + j * 16
                    v = idbuf[pl.ds(o, 16)]
                    v = jnp.minimum(jnp.maximum(v, 0), _VS[c] - 1) + _OFF[c]
                    idbuf[pl.ds(o, 16)] = v
                    return 0

                lax.fori_loop(0, _C // 16, tbody, 0)
            copies = [
                pltpu.async_copy(emb_hbm.at[idbuf.at[pl.ds(g * _G, _G)]],
                                 rows.at[pl.ds(g * _G, _G)], gsem)
                for g in range(_NG)
            ]
            for cp in copies:
                cp.wait()

            def abody(t, _):
                for f in range(_D // 16):
                    fo = f * 16
                    acc = svec[0] * rows[t, pl.ds(fo, 16)]
                    for c in range(1, _NCOMP):
                        acc = acc + svec[c] * rows[c * _C + t, pl.ds(fo, 16)]
                    wbuf[t, pl.ds(fo, 16)] = acc
                return 0

            lax.fori_loop(0, _C, abody, 0)
            pltpu.sync_copy(wbuf, w_hbm.at[pl.ds(tb, _C)])

    return body(ids, emb_weight, scales_b)


def _tc_up_proj(w, up_t):
    tb = 2048

    def mm(w_ref, u_ref, o_ref):
        o_ref[...] = jnp.dot(w_ref[...], u_ref[...],
                             preferred_element_type=jnp.float32)

    return pl.pallas_call(
        mm,
        grid=(_T // tb,),
        in_specs=[
            pl.BlockSpec((tb, _D), lambda i: (i, 0)),
            pl.BlockSpec((_D, _H), lambda i: (0, 0)),
        ],
        out_specs=pl.BlockSpec((tb, _H), lambda i: (i, 0)),
        out_shape=jax.ShapeDtypeStruct((_T, _H), jnp.float32),
        compiler_params=pltpu.CompilerParams(
            dimension_semantics=("arbitrary",)),
    )(w, up_t)


def kernel(structure_ids, dep_levels, ast_depth_ids, sibling_index_ids,
           node_type_ids, emb_weight, up_proj_weight, component_scales):
    ids = jnp.concatenate(
        [a.reshape(-1) for a in (structure_ids, dep_levels, ast_depth_ids,
                                 sibling_index_ids, node_type_ids)], axis=0)
    scales_b = jnp.broadcast_to(
        component_scales.reshape(_NCOMP, 1).astype(jnp.float32), (_NCOMP, 16))
    w = _sc_weighted(ids, emb_weight, scales_b)
    out = _tc_up_proj(w, up_proj_weight.T)
    return out.reshape(_B, _S, _H)
